# Initial kernel scaffold; baseline (speedup 1.0000x reference)
import jax
import jax.numpy as jnp
from jax.experimental import pallas as pl


def kernel(x, edge_index, edge_weight, h, c, Wz, bz, Wr, br, Wh, bh, Wl, bl):
    raise NotImplementedError("write your pallas kernel here")


# speedup vs baseline: 6.1592x; 6.1592x over previous
import functools

import jax
import jax.numpy as jnp
from jax import lax
from jax.experimental import pallas as pl
from jax.experimental.pallas import tpu as pltpu
import jax.experimental.pallas.tpu_sc as plsc

N = 50000
E = 800000
F_IN = 35
F_OUT = 32

NP = 50176
FP = 40
EP = 819200
CHK = 64
NSUB = 16
ROWS_PER_SUB = NP // NSUB
EDGES_PER_SUB = EP // NSUB
NCHUNK = EDGES_PER_SUB // CHK
OCH = 64
BN = 6272
GRID = NP // BN

_MESH = dict(core_axis_name="c", subcore_axis_name="s", num_cores=2,
             num_subcores=NSUB)


def _deg_call(ei_flat, ew):
    @functools.partial(
        pl.kernel,
        out_type=jax.ShapeDtypeStruct((2 * NP,), jnp.float32),
        mesh=plsc.VectorSubcoreMesh(**_MESH),
        scratch_types=[
            pltpu.VMEM_SHARED((NP,), jnp.float32),
            pltpu.VMEM((CHK,), jnp.int32),
            pltpu.VMEM((CHK,), jnp.float32),
            pltpu.VMEM((ROWS_PER_SUB,), jnp.float32),
        ],
    )
    def deg_kernel(ei_hbm, ew_hbm, deg_hbm, acc_sh, idx_v, ew_v, buf_v):
        c = lax.axis_index("c")
        s = lax.axis_index("s")

        def zero_body(i, carry):
            buf_v[pl.ds(i * 16, 16)] = jnp.zeros((16,), jnp.float32)
            return carry

        lax.fori_loop(0, ROWS_PER_SUB // 16, zero_body, 0)
        pltpu.sync_copy(buf_v, acc_sh.at[pl.ds(s * ROWS_PER_SUB, ROWS_PER_SUB)])
        plsc.subcore_barrier()

        def chunk_body(g, carry):
            base = s * EDGES_PER_SUB + g * CHK
            pltpu.sync_copy(ei_hbm.at[pl.ds(c * EP + base, CHK)], idx_v)
            pltpu.sync_copy(ew_hbm.at[pl.ds(base, CHK)], ew_v)
            pltpu.sync_copy(ew_v, acc_sh.at[idx_v], add=True)
            return carry

        lax.fori_loop(0, NCHUNK, chunk_body, 0)
        plsc.subcore_barrier()
        pltpu.sync_copy(acc_sh.at[pl.ds(s * ROWS_PER_SUB, ROWS_PER_SUB)], buf_v)
        pltpu.sync_copy(buf_v,
                        deg_hbm.at[pl.ds(c * NP + s * ROWS_PER_SUB,
                                         ROWS_PER_SUB)])

    return deg_kernel(ei_flat, ew)


def _spmm_call(src_flat, rowadj, col, ew):
    @functools.partial(
        pl.kernel,
        out_type=jax.ShapeDtypeStruct((2 * NP, FP), jnp.float32),
        mesh=plsc.VectorSubcoreMesh(**_MESH),
        compiler_params=pltpu.CompilerParams(use_tc_tiling_on_sc=False),
        scratch_types=[
            pltpu.VMEM_SHARED((NP, FP), jnp.float32),
            pltpu.VMEM((CHK,), jnp.int32),
            pltpu.VMEM((CHK,), jnp.int32),
            pltpu.VMEM((CHK,), jnp.float32),
            pltpu.VMEM((CHK, FP), jnp.float32),
            pltpu.VMEM((CHK, FP), jnp.float32),
            pltpu.SemaphoreType.DMA,
        ],
    )
    def spmm_kernel(src_hbm, rowadj_hbm, col_hbm, ew_hbm, out_hbm,
                    acc_sh, row_v, col_v, ew_v, dat_v, sc_v, sem):
        c = lax.axis_index("c")
        s = lax.axis_index("s")
        zero16 = jnp.zeros((16,), jnp.float32)

        for r in range(CHK):
            sc_v[r, pl.ds(0, 16)] = zero16
            sc_v[r, pl.ds(16, 16)] = zero16
            sc_v[r, pl.ds(24, 16)] = zero16

        def zero_dma(k, carry):
            pltpu.sync_copy(
                sc_v.at[pl.ds(0, OCH), :],
                acc_sh.at[pl.ds(s * ROWS_PER_SUB + k * OCH, OCH), :])
            return carry

        lax.fori_loop(0, ROWS_PER_SUB // OCH, zero_dma, 0)
        plsc.subcore_barrier()

        def chunk_body(g, carry):
            base = s * EDGES_PER_SUB + g * CHK
            pltpu.sync_copy(rowadj_hbm.at[pl.ds(c * EP + base, CHK)], row_v)
            pltpu.sync_copy(col_hbm.at[pl.ds(base, CHK)], col_v)
            pltpu.sync_copy(ew_hbm.at[pl.ds(base, CHK)], ew_v)
            pltpu.async_copy(src_hbm.at[row_v], dat_v, sem).wait()
            for jg in range(CHK // 16):
                ew16 = ew_v[pl.ds(jg * 16, 16)]
                for rr in range(16):
                    r = jg * 16 + rr
                    m = jnp.take_along_axis(
                        ew16, jnp.full((16,), rr, jnp.int32), axis=0)
                    sc_v[r, pl.ds(0, 16)] = dat_v[r, pl.ds(0, 16)] * m
                    sc_v[r, pl.ds(16, 16)] = dat_v[r, pl.ds(16, 16)] * m
                    sc_v[r, pl.ds(24, 16)] = dat_v[r, pl.ds(24, 16)] * m
            pltpu.sync_copy(sc_v, acc_sh.at[col_v], add=True)
            return carry

        lax.fori_loop(0, NCHUNK, chunk_body, 0)
        plsc.subcore_barrier()

        def out_dma(k, carry):
            b = s * ROWS_PER_SUB + k * OCH
            pltpu.sync_copy(acc_sh.at[pl.ds(b, OCH), :],
                            dat_v.at[pl.ds(0, OCH), :])
            pltpu.sync_copy(dat_v.at[pl.ds(0, OCH), :],
                            out_hbm.at[pl.ds(c * NP + b, OCH), :])
            return carry

        lax.fori_loop(0, ROWS_PER_SUB // OCH, out_dma, 0)

    return spmm_kernel(src_flat, rowadj, col, ew)


def _recip_body(degp_ref, x_ref, ro_ref, ri_ref, st1_ref):
    dp = degp_ref[...]
    ro = jnp.where(dp[0] > 0, 1.0 / dp[0], 0.0)
    ri = jnp.where(dp[1] > 0, 1.0 / dp[1], 0.0)
    xb = x_ref[...]
    ro_ref[...] = ro[:, None]
    ri_ref[...] = ri[:, None]
    st1_ref[0] = xb * ro[:, None]
    st1_ref[1] = xb


def _recip_call(degp, x36):
    return pl.pallas_call(
        _recip_body,
        grid=(GRID,),
        in_specs=[
            pl.BlockSpec((2, BN), lambda i: (0, i)),
            pl.BlockSpec((BN, FP), lambda i: (i, 0)),
        ],
        out_specs=[
            pl.BlockSpec((BN, 1), lambda i: (i, 0)),
            pl.BlockSpec((BN, 1), lambda i: (i, 0)),
            pl.BlockSpec((2, BN, FP), lambda i: (0, i, 0)),
        ],
        out_shape=[
            jax.ShapeDtypeStruct((NP, 1), jnp.float32),
            jax.ShapeDtypeStruct((NP, 1), jnp.float32),
            jax.ShapeDtypeStruct((2, NP, FP), jnp.float32),
        ],
    )(degp, x36)


def _rescale_body(u1_ref, ro_ref, ri_ref, st2_ref):
    st2_ref[0] = u1_ref[0] * ro_ref[...]
    st2_ref[1] = u1_ref[1] * ri_ref[...]


def _rescale_call(u1, ro, ri):
    return pl.pallas_call(
        _rescale_body,
        grid=(GRID,),
        in_specs=[
            pl.BlockSpec((2, BN, FP), lambda i: (0, i, 0)),
            pl.BlockSpec((BN, 1), lambda i: (i, 0)),
            pl.BlockSpec((BN, 1), lambda i: (i, 0)),
        ],
        out_specs=pl.BlockSpec((2, BN, FP), lambda i: (0, i, 0)),
        out_shape=jax.ShapeDtypeStruct((2, NP, FP), jnp.float32),
    )(u1, ro, ri)


def _head_body(x_ref, u1_ref, bi_ref, u2_ref, ri_ref, wst_ref, bcat_ref,
               wl_ref, bl_ref, y_ref):
    xb = x_ref[...]
    f2 = u1_ref[0]
    f3 = bi_ref[0]
    f4 = u2_ref[0]
    f5 = u2_ref[1] * ri_ref[...]
    w = wst_ref[...]
    g = (jnp.dot(xb, w[0], preferred_element_type=jnp.float32)
         + jnp.dot(f2, w[1], preferred_element_type=jnp.float32)
         + jnp.dot(f3, w[2], preferred_element_type=jnp.float32)
         + jnp.dot(f4, w[3], preferred_element_type=jnp.float32)
         + jnp.dot(f5, w[4], preferred_element_type=jnp.float32)
         + bcat_ref[...])
    z = jax.nn.sigmoid(g[:, :F_OUT])
    ht = jnp.tanh(g[:, F_OUT:])
    hcell = (1.0 - z) * ht
    y_ref[...] = (jnp.dot(jnp.maximum(hcell, 0.0), wl_ref[...],
                          preferred_element_type=jnp.float32) + bl_ref[...])


def _head_call(x36, u1, st2, u2, ri, wst, bcat, wl, bl):
    return pl.pallas_call(
        _head_body,
        grid=(GRID,),
        in_specs=[
            pl.BlockSpec((BN, FP), lambda i: (i, 0)),
            pl.BlockSpec((2, BN, FP), lambda i: (0, i, 0)),
            pl.BlockSpec((1, BN, FP), lambda i: (1, i, 0)),
            pl.BlockSpec((2, BN, FP), lambda i: (0, i, 0)),
            pl.BlockSpec((BN, 1), lambda i: (i, 0)),
            pl.BlockSpec((5, FP, 2 * F_OUT), lambda i: (0, 0, 0)),
            pl.BlockSpec((1, 2 * F_OUT), lambda i: (0, 0)),
            pl.BlockSpec((F_OUT, 1), lambda i: (0, 0)),
            pl.BlockSpec((1, 1), lambda i: (0, 0)),
        ],
        out_specs=pl.BlockSpec((BN, 1), lambda i: (i, 0)),
        out_shape=jax.ShapeDtypeStruct((NP, 1), jnp.float32),
    )(x36, u1, st2, u2, ri, wst, bcat, wl, bl)


def kernel(x, edge_index, edge_weight, h, c, Wz, bz, Wr, br, Wh, bh, Wl, bl):
    x36 = jnp.pad(x, ((0, NP - N), (0, FP - F_IN)))
    ei = jnp.pad(edge_index, ((0, 0), (0, EP - E)))
    ew = jnp.pad(edge_weight, ((0, EP - E),))
    ei_flat = ei.reshape(2 * EP)

    deg2 = _deg_call(ei_flat, ew)
    ro, ri, st1 = _recip_call(deg2.reshape(2, NP), x36)
    rowadj = jnp.concatenate([ei[0], ei[0] + NP])
    u1 = _spmm_call(st1.reshape(2 * NP, FP), rowadj, ei[1], ew).reshape(2, NP, FP)
    st2 = _rescale_call(u1, ro, ri)
    u2 = _spmm_call(st2.reshape(2 * NP, FP), rowadj, ei[1], ew).reshape(2, NP, FP)

    def combos(W):
        Wc = jnp.pad(W[:, :, :F_IN, :], ((0, 0), (0, 0), (0, FP - F_IN), (0, 0)))
        return [Wc[0, 0] + Wc[1, 0] - Wc[0, 2] - Wc[1, 2],
                Wc[0, 1], Wc[1, 1], 2.0 * Wc[0, 2], 2.0 * Wc[1, 2]]

    wst = jnp.stack([jnp.concatenate([a, b], axis=1)
                     for a, b in zip(combos(Wz), combos(Wh))])
    bcat = jnp.concatenate([bz, bh]).reshape(1, 2 * F_OUT)

    y = _head_call(x36, u1, st2, u2, ri, wst, bcat, Wl, bl.reshape(1, 1))
    return y[:N]
